# baseline (speedup 1.0000x reference)
import functools

import jax
import jax.numpy as jnp
from jax.experimental import pallas as pl

_BM = 1024
_BK = 1024


def _dist_body(x_ref, c_ref, o_ref):
    xb = x_ref[...]
    cb = c_ref[...]
    prod = jax.lax.dot_general(
        xb, cb, (((1,), (1,)), ((), ())),
        preferred_element_type=jnp.float32)
    x2 = jnp.sum(xb * xb, axis=1, keepdims=True)
    c2 = jnp.sum(cb * cb, axis=1)[None, :]
    o_ref[...] = x2 + c2 - 2.0 * prod


@functools.partial(jax.jit, static_argnames=())
def kernel(x, Ck):
    Bx, Sx, Dx = x.shape
    feats = x.reshape(Bx * Sx, Dx)
    C = Ck.reshape(Ck.shape[1], Dx)
    M, K = feats.shape[0], C.shape[0]
    grid = (M // _BM, K // _BK)
    out = pl.pallas_call(
        _dist_body,
        grid=grid,
        in_specs=[
            pl.BlockSpec((_BM, Dx), lambda i, j: (i, 0)),
            pl.BlockSpec((_BK, Dx), lambda i, j: (j, 0)),
        ],
        out_specs=pl.BlockSpec((_BM, _BK), lambda i, j: (i, j)),
        out_shape=jax.ShapeDtypeStruct((M, K), jnp.float32),
    )(feats, C)
    return out.reshape(Bx, Sx, K)


# speedup vs baseline: 1.2934x; 1.2934x over previous
import functools

import jax
import jax.numpy as jnp
from jax.experimental import pallas as pl

_BK = 1024


def _dist_body(x_ref, c_ref, o_ref):
    xb = x_ref[...]
    cb = c_ref[...]
    prod = jax.lax.dot_general(
        xb, cb, (((1,), (1,)), ((), ())),
        preferred_element_type=jnp.float32)
    x2 = jnp.sum(xb * xb, axis=1, keepdims=True)
    c2 = jnp.sum(cb * cb, axis=1)[None, :]
    o_ref[...] = x2 + c2 - 2.0 * prod


@functools.partial(jax.jit, static_argnames=())
def kernel(x, Ck):
    Bx, Sx, Dx = x.shape
    feats = x.reshape(Bx * Sx, Dx)
    C = Ck.reshape(Ck.shape[1], Dx)
    M, K = feats.shape[0], C.shape[0]
    grid = (K // _BK,)
    out = pl.pallas_call(
        _dist_body,
        grid=grid,
        in_specs=[
            pl.BlockSpec((M, Dx), lambda j: (0, 0)),
            pl.BlockSpec((_BK, Dx), lambda j: (j, 0)),
        ],
        out_specs=pl.BlockSpec((M, _BK), lambda j: (0, j)),
        out_shape=jax.ShapeDtypeStruct((M, K), jnp.float32),
    )(feats, C)
    return out.reshape(Bx, Sx, K)


# speedup vs baseline: 1.2975x; 1.0032x over previous
import functools

import jax
import jax.numpy as jnp
from jax.experimental import pallas as pl

_BM = 512


def _dist_body(x_ref, c_ref, o_ref):
    xb = x_ref[...]
    cb = c_ref[...]
    prod = jax.lax.dot_general(
        xb, cb, (((1,), (1,)), ((), ())),
        preferred_element_type=jnp.float32)
    x2 = jnp.sum(xb * xb, axis=1, keepdims=True)
    c2 = jnp.sum(cb * cb, axis=1)[None, :]
    o_ref[...] = x2 + c2 - 2.0 * prod


@functools.partial(jax.jit, static_argnames=())
def kernel(x, Ck):
    Bx, Sx, Dx = x.shape
    feats = x.reshape(Bx * Sx, Dx)
    C = Ck.reshape(Ck.shape[1], Dx)
    M, K = feats.shape[0], C.shape[0]
    grid = (M // _BM,)
    out = pl.pallas_call(
        _dist_body,
        grid=grid,
        in_specs=[
            pl.BlockSpec((_BM, Dx), lambda i: (i, 0)),
            pl.BlockSpec((K, Dx), lambda i: (0, 0)),
        ],
        out_specs=pl.BlockSpec((_BM, K), lambda i: (i, 0)),
        out_shape=jax.ShapeDtypeStruct((M, K), jnp.float32),
    )(feats, C)
    return out.reshape(Bx, Sx, K)


# speedup vs baseline: 1.3001x; 1.0020x over previous
import functools

import jax
import jax.numpy as jnp
from jax.experimental import pallas as pl
from jax.experimental.pallas import tpu as pltpu

_BM = 512


def _dist_body(x_ref, c_ref, o_ref):
    xb = x_ref[...]
    cb = c_ref[...]
    prod = jax.lax.dot_general(
        xb, cb, (((1,), (1,)), ((), ())),
        preferred_element_type=jnp.float32)
    x2 = jnp.sum(xb * xb, axis=1, keepdims=True)
    c2 = jnp.sum(cb * cb, axis=1)[None, :]
    o_ref[...] = x2 + c2 - 2.0 * prod


@functools.partial(jax.jit, static_argnames=())
def kernel(x, Ck):
    Bx, Sx, Dx = x.shape
    feats = x.reshape(Bx * Sx, Dx)
    C = Ck.reshape(Ck.shape[1], Dx)
    M, K = feats.shape[0], C.shape[0]
    grid = (M // _BM,)
    out = pl.pallas_call(
        _dist_body,
        grid=grid,
        in_specs=[
            pl.BlockSpec((_BM, Dx), lambda i: (i, 0)),
            pl.BlockSpec((K, Dx), lambda i: (0, 0)),
        ],
        out_specs=pl.BlockSpec((_BM, K), lambda i: (i, 0)),
        out_shape=jax.ShapeDtypeStruct((M, K), jnp.float32),
        compiler_params=pltpu.CompilerParams(
            dimension_semantics=("parallel",)),
    )(feats, C)
    return out.reshape(Bx, Sx, K)
